# baseline (speedup 1.0000x reference)
import functools

import jax
import jax.numpy as jnp
from jax import lax
from jax.experimental import pallas as pl
from jax.experimental.pallas import tpu as pltpu
from jax.experimental.pallas import tpu_sc as plsc

B, L, V, H = 1024, 200, 1000, 512
N_SPK, SPK_U = 128, 64
EPS = 1e-05

NC, NS = 2, 16
NW = NC * NS
TOKENS = B * L

SC_TOKENS = 102400
TC_TOKENS = TOKENS - SC_TOKENS
TC_BLK = 512

PER_W = SC_TOKENS // NW
CHUNK = 64
N_CHUNKS = PER_W // CHUNK


def _prep_body(emb_ref, g_ref, b_ref, spk_ids_ref, spk_tab_ref, fc_w_ref,
               fc_b_ref, ln_out_ref, hi_out_ref, lo_out_ref, spk_out_ref):
    x = emb_ref[...]
    mean = jnp.mean(x, axis=1, keepdims=True)
    xc = x - mean
    var = jnp.mean(xc * xc, axis=1, keepdims=True)
    ln = (xc * lax.rsqrt(var + EPS) * g_ref[0, :][None, :]
          + b_ref[0, :][None, :])
    ln_out_ref[...] = ln
    hi = ln.astype(jnp.bfloat16)
    hi_out_ref[...] = hi
    lo_out_ref[...] = (ln - hi.astype(jnp.float32)).astype(jnp.bfloat16)
    sid = spk_ids_ref[...]
    onehot = (sid == lax.broadcasted_iota(jnp.int32, (B, N_SPK), 1))
    se = jnp.dot(onehot.astype(jnp.float32), spk_tab_ref[...],
                 preferred_element_type=jnp.float32)
    feat = jnp.dot(se, fc_w_ref[...], preferred_element_type=jnp.float32)
    feat = feat + fc_b_ref[0, :][None, :]
    spk_out_ref[...] = jax.nn.softplus(feat)


_prep = pl.pallas_call(
    _prep_body,
    out_shape=[
        jax.ShapeDtypeStruct((V, H), jnp.float32),
        jax.ShapeDtypeStruct((V, H), jnp.bfloat16),
        jax.ShapeDtypeStruct((V, H), jnp.bfloat16),
        jax.ShapeDtypeStruct((B, H), jnp.float32),
    ],
)


def _tc_gather_body(ids_ref, hi_ref, lo_ref, out_ref):
    ids = ids_ref[...]
    oh = (ids == lax.broadcasted_iota(jnp.int32, (TC_BLK, V), 1))
    oh = oh.astype(jnp.bfloat16)
    acc = jnp.dot(oh, hi_ref[...], preferred_element_type=jnp.float32)
    acc = acc + jnp.dot(oh, lo_ref[...], preferred_element_type=jnp.float32)
    out_ref[...] = acc


_tc_gather = pl.pallas_call(
    _tc_gather_body,
    grid=(TC_TOKENS // TC_BLK,),
    in_specs=[
        pl.BlockSpec((TC_BLK, 1), lambda i: (i, 0)),
        pl.BlockSpec((V, H), lambda i: (0, 0)),
        pl.BlockSpec((V, H), lambda i: (0, 0)),
    ],
    out_specs=pl.BlockSpec((TC_BLK, H), lambda i: (i, 0)),
    out_shape=jax.ShapeDtypeStruct((TC_TOKENS, H), jnp.float32),
)


_sc_mesh = plsc.VectorSubcoreMesh(core_axis_name="c", subcore_axis_name="s")


@functools.partial(
    pl.kernel,
    out_type=jax.ShapeDtypeStruct((SC_TOKENS, H), jnp.float32),
    mesh=_sc_mesh,
    scratch_types=[
        pltpu.VMEM((PER_W,), jnp.int32),
        pltpu.VMEM((CHUNK, H), jnp.float32),
        pltpu.VMEM((CHUNK, H), jnp.float32),
        pltpu.SemaphoreType.DMA,
        pltpu.SemaphoreType.DMA,
        pltpu.SemaphoreType.DMA,
        pltpu.SemaphoreType.DMA,
    ],
)
def _sc_gather(table_hbm, idx_hbm, out_hbm, idx_v, buf0, buf1,
               gs0, gs1, os0, os1):
    wid = lax.axis_index("s") * NC + lax.axis_index("c")
    base = pl.multiple_of(wid * PER_W, PER_W)
    pltpu.sync_copy(idx_hbm.at[pl.ds(base, PER_W)], idx_v)

    def gat(c, buf, sem):
        off = pl.multiple_of(c * CHUNK, CHUNK)
        return pltpu.make_async_copy(
            table_hbm.at[idx_v.at[pl.ds(off, CHUNK)]], buf, sem)

    def put(c, buf, sem):
        off = pl.multiple_of(c * CHUNK, CHUNK)
        return pltpu.make_async_copy(buf, out_hbm.at[pl.ds(base + off, CHUNK)],
                                     sem)

    gat(0, buf0, gs0).start()
    gat(1, buf1, gs1).start()

    @pl.loop(0, N_CHUNKS - 2, step=2)
    def _(j):
        gat(j, buf0, gs0).wait()
        put(j, buf0, os0).start()
        gat(j + 1, buf1, gs1).wait()
        put(j + 1, buf1, os1).start()
        put(j, buf0, os0).wait()
        gat(j + 2, buf0, gs0).start()
        put(j + 1, buf1, os1).wait()
        gat(j + 3, buf1, gs1).start()

    gat(N_CHUNKS - 2, buf0, gs0).wait()
    put(N_CHUNKS - 2, buf0, os0).start()
    gat(N_CHUNKS - 1, buf1, gs1).wait()
    put(N_CHUNKS - 1, buf1, os1).start()
    put(N_CHUNKS - 2, buf0, os0).wait()
    put(N_CHUNKS - 1, buf1, os1).wait()


def kernel(input_ids, speaker_ids, char_emb, spk_table, fc_w, fc_b, ln_gamma,
           ln_beta):
    ln_table, hi, lo, spk_feat = _prep(char_emb, ln_gamma.reshape(1, H),
                                       ln_beta.reshape(1, H), speaker_ids,
                                       spk_table, fc_w, fc_b.reshape(1, H))
    flat = input_ids.reshape(TOKENS)
    sc_out = _sc_gather(ln_table, flat[:SC_TOKENS])
    tc_out = _tc_gather(flat[SC_TOKENS:].reshape(TC_TOKENS, 1), hi, lo)
    emb = jnp.concatenate([sc_out, tc_out], axis=0).reshape(B, L, H)
    return emb, spk_feat.reshape(B, 1, H)


# speedup vs baseline: 1.8956x; 1.8956x over previous
import functools

import jax
import jax.numpy as jnp
from jax import lax
from jax.experimental import pallas as pl
from jax.experimental.pallas import tpu as pltpu
from jax.experimental.pallas import tpu_sc as plsc

B, L, V, H = 1024, 200, 1000, 512
N_SPK, SPK_U = 128, 64
EPS = 1e-05

NC, NS = 2, 16
NW = NC * NS
TOKENS = B * L
PER_W = TOKENS // NW
CHUNK = 40
N_CHUNKS = PER_W // CHUNK
NBUF = 4


def _prep_body(emb_ref, g_ref, b_ref, spk_ids_ref, spk_tab_ref, fc_w_ref,
               fc_b_ref, ln_out_ref, spk_out_ref):
    x = emb_ref[...]
    mean = jnp.mean(x, axis=1, keepdims=True)
    xc = x - mean
    var = jnp.mean(xc * xc, axis=1, keepdims=True)
    ln_out_ref[...] = (xc * lax.rsqrt(var + EPS) * g_ref[0, :][None, :]
                       + b_ref[0, :][None, :])
    sid = spk_ids_ref[...]
    onehot = (sid == lax.broadcasted_iota(jnp.int32, (B, N_SPK), 1))
    se = jnp.dot(onehot.astype(jnp.float32), spk_tab_ref[...],
                 preferred_element_type=jnp.float32)
    feat = jnp.dot(se, fc_w_ref[...], preferred_element_type=jnp.float32)
    feat = feat + fc_b_ref[0, :][None, :]
    spk_out_ref[...] = jax.nn.softplus(feat)


_prep = pl.pallas_call(
    _prep_body,
    out_shape=[
        jax.ShapeDtypeStruct((V, H), jnp.float32),
        jax.ShapeDtypeStruct((B, H), jnp.float32),
    ],
)


_sc_mesh = plsc.VectorSubcoreMesh(core_axis_name="c", subcore_axis_name="s")


@functools.partial(
    pl.kernel,
    out_type=jax.ShapeDtypeStruct((TOKENS, H), jnp.float32),
    mesh=_sc_mesh,
    scratch_types=(
        [pltpu.VMEM((PER_W,), jnp.int32)]
        + [pltpu.VMEM((CHUNK, H), jnp.float32)] * NBUF
        + [pltpu.SemaphoreType.DMA] * (2 * NBUF)
    ),
)
def _sc_gather(table_hbm, idx_hbm, out_hbm, idx_v, *bufs_and_sems):
    bufs = bufs_and_sems[:NBUF]
    gsems = bufs_and_sems[NBUF:2 * NBUF]
    osems = bufs_and_sems[2 * NBUF:]
    wid = lax.axis_index("s") * NC + lax.axis_index("c")
    base = pl.multiple_of(wid * PER_W, PER_W)
    pltpu.sync_copy(idx_hbm.at[pl.ds(base, PER_W)], idx_v)

    def gat(c, b):
        off = pl.multiple_of(c * CHUNK, CHUNK)
        return pltpu.make_async_copy(
            table_hbm.at[idx_v.at[pl.ds(off, CHUNK)]], bufs[b], gsems[b])

    def put(c, b):
        off = pl.multiple_of(c * CHUNK, CHUNK)
        return pltpu.make_async_copy(bufs[b],
                                     out_hbm.at[pl.ds(base + off, CHUNK)],
                                     osems[b])

    for b in range(NBUF):
        gat(b, b).start()

    @pl.loop(0, N_CHUNKS, step=NBUF)
    def _(j):
        for b in range(NBUF):
            gat(j + b, b).wait()
            put(j + b, b).start()
        for b in range(NBUF):
            put(j + b, b).wait()
            nxt = j + NBUF + b

            @pl.when(nxt < N_CHUNKS)
            def _():
                gat(nxt, b).start()


def kernel(input_ids, speaker_ids, char_emb, spk_table, fc_w, fc_b, ln_gamma,
           ln_beta):
    ln_table, spk_feat = _prep(char_emb, ln_gamma.reshape(1, H),
                               ln_beta.reshape(1, H), speaker_ids, spk_table,
                               fc_w, fc_b.reshape(1, H))
    flat = _sc_gather(ln_table, input_ids.reshape(TOKENS))
    return flat.reshape(B, L, H), spk_feat.reshape(B, 1, H)


# speedup vs baseline: 1.9282x; 1.0172x over previous
import functools

import jax
import jax.numpy as jnp
from jax import lax
from jax.experimental import pallas as pl
from jax.experimental.pallas import tpu as pltpu
from jax.experimental.pallas import tpu_sc as plsc

B, L, V, H = 1024, 200, 1000, 512
N_SPK, SPK_U = 128, 64
EPS = 1e-05

NC, NS = 2, 16
NW = NC * NS
TOKENS = B * L
PER_W = TOKENS // NW
CHUNK = 40
N_CHUNKS = PER_W // CHUNK
NBUF = 4


def _prep_body(emb_ref, g_ref, b_ref, spk_ids_ref, spk_tab_ref, fc_w_ref,
               fc_b_ref, ln_out_ref, spk_out_ref):
    x = emb_ref[...]
    mean = jnp.mean(x, axis=1, keepdims=True)
    xc = x - mean
    var = jnp.mean(xc * xc, axis=1, keepdims=True)
    ln_out_ref[...] = (xc * lax.rsqrt(var + EPS) * g_ref[0, :][None, :]
                       + b_ref[0, :][None, :])
    sid = spk_ids_ref[...]
    onehot = (sid == lax.broadcasted_iota(jnp.int32, (B, N_SPK), 1))
    se = jnp.dot(onehot.astype(jnp.float32), spk_tab_ref[...],
                 preferred_element_type=jnp.float32)
    feat = jnp.dot(se, fc_w_ref[...], preferred_element_type=jnp.float32)
    feat = feat + fc_b_ref[0, :][None, :]
    spk_out_ref[...] = jax.nn.softplus(feat)


_prep = pl.pallas_call(
    _prep_body,
    out_shape=[
        jax.ShapeDtypeStruct((V, H), jnp.float32),
        jax.ShapeDtypeStruct((B, H), jnp.float32),
    ],
)


_sc_mesh = plsc.VectorSubcoreMesh(core_axis_name="c", subcore_axis_name="s")


@functools.partial(
    pl.kernel,
    out_type=jax.ShapeDtypeStruct((TOKENS, H), jnp.float32),
    mesh=_sc_mesh,
    scratch_types=(
        [pltpu.VMEM((PER_W,), jnp.int32)]
        + [pltpu.VMEM((CHUNK, H), jnp.float32)] * NBUF
        + [pltpu.SemaphoreType.DMA] * (2 * NBUF)
    ),
)
def _sc_gather(table_hbm, idx_hbm, out_hbm, idx_v, *bufs_and_sems):
    bufs = bufs_and_sems[:NBUF]
    gsems = bufs_and_sems[NBUF:2 * NBUF]
    osems = bufs_and_sems[2 * NBUF:]
    sid = lax.axis_index("s")
    wid = sid * NC + lax.axis_index("c")
    base = pl.multiple_of(wid * PER_W, PER_W)
    pltpu.sync_copy(idx_hbm.at[pl.ds(base, PER_W)], idx_v)

    def gat(c, b):
        off = pl.multiple_of(c * CHUNK, CHUNK)
        return pltpu.make_async_copy(
            table_hbm.at[idx_v.at[pl.ds(off, CHUNK)]], bufs[b], gsems[b])

    def put(c, b):
        off = pl.multiple_of(c * CHUNK, CHUNK)
        return pltpu.make_async_copy(bufs[b],
                                     out_hbm.at[pl.ds(base + off, CHUNK)],
                                     osems[b])

    for b in range(2):
        gat(b, b).start()

    @pl.loop(0, N_CHUNKS, step=NBUF)
    def _(j):
        for b in range(NBUF):
            c = j + b
            gat(c, b).wait()
            put(c, b).start()
            d = (b + 2) % NBUF
            cm2 = c - 2

            @pl.when(cm2 >= 0)
            def _():
                put(cm2, d).wait()

            cp2 = c + 2

            @pl.when(cp2 < N_CHUNKS)
            def _():
                gat(cp2, d).start()

    put(N_CHUNKS - 2, (N_CHUNKS - 2) % NBUF).wait()
    put(N_CHUNKS - 1, (N_CHUNKS - 1) % NBUF).wait()


def kernel(input_ids, speaker_ids, char_emb, spk_table, fc_w, fc_b, ln_gamma,
           ln_beta):
    ln_table, spk_feat = _prep(char_emb, ln_gamma.reshape(1, H),
                               ln_beta.reshape(1, H), speaker_ids, spk_table,
                               fc_w, fc_b.reshape(1, H))
    flat = _sc_gather(ln_table, input_ids.reshape(TOKENS))
    return flat.reshape(B, L, H), spk_feat.reshape(B, 1, H)


# speedup vs baseline: 1.9288x; 1.0003x over previous
import functools

import jax
import jax.numpy as jnp
from jax import lax
from jax.experimental import pallas as pl
from jax.experimental.pallas import tpu as pltpu
from jax.experimental.pallas import tpu_sc as plsc

B, L, V, H = 1024, 200, 1000, 512
N_SPK, SPK_U = 128, 64
EPS = 1e-05

NC, NS = 2, 16
NW = NC * NS
TOKENS = B * L
PER_W = TOKENS // NW
CHUNK = 40
N_CHUNKS = PER_W // CHUNK
NBUF = 5
LAG = 2
FREE = NBUF - LAG


def _prep_body(emb_ref, g_ref, b_ref, spk_ids_ref, spk_tab_ref, fc_w_ref,
               fc_b_ref, ln_out_ref, spk_out_ref):
    x = emb_ref[...]
    mean = jnp.mean(x, axis=1, keepdims=True)
    xc = x - mean
    var = jnp.mean(xc * xc, axis=1, keepdims=True)
    ln_out_ref[...] = (xc * lax.rsqrt(var + EPS) * g_ref[0, :][None, :]
                       + b_ref[0, :][None, :])
    sid = spk_ids_ref[...]
    onehot = (sid == lax.broadcasted_iota(jnp.int32, (B, N_SPK), 1))
    se = jnp.dot(onehot.astype(jnp.float32), spk_tab_ref[...],
                 preferred_element_type=jnp.float32)
    feat = jnp.dot(se, fc_w_ref[...], preferred_element_type=jnp.float32)
    feat = feat + fc_b_ref[0, :][None, :]
    spk_out_ref[...] = jax.nn.softplus(feat)


_prep = pl.pallas_call(
    _prep_body,
    out_shape=[
        jax.ShapeDtypeStruct((V, H), jnp.float32),
        jax.ShapeDtypeStruct((B, H), jnp.float32),
    ],
)


_sc_mesh = plsc.VectorSubcoreMesh(core_axis_name="c", subcore_axis_name="s")


@functools.partial(
    pl.kernel,
    out_type=jax.ShapeDtypeStruct((TOKENS, H), jnp.float32),
    mesh=_sc_mesh,
    scratch_types=(
        [pltpu.VMEM((PER_W,), jnp.int32)]
        + [pltpu.VMEM((CHUNK, H), jnp.float32)] * NBUF
        + [pltpu.SemaphoreType.DMA] * (2 * NBUF)
    ),
)
def _sc_gather(table_hbm, idx_hbm, out_hbm, idx_v, *bufs_and_sems):
    bufs = bufs_and_sems[:NBUF]
    gsems = bufs_and_sems[NBUF:2 * NBUF]
    osems = bufs_and_sems[2 * NBUF:]
    sid = lax.axis_index("s")
    wid = sid * NC + lax.axis_index("c")
    base = pl.multiple_of(wid * PER_W, PER_W)
    pltpu.sync_copy(idx_hbm.at[pl.ds(base, PER_W)], idx_v)

    def gat(c, b):
        off = pl.multiple_of(c * CHUNK, CHUNK)
        return pltpu.make_async_copy(
            table_hbm.at[idx_v.at[pl.ds(off, CHUNK)]], bufs[b], gsems[b])

    def put(c, b):
        off = pl.multiple_of(c * CHUNK, CHUNK)
        return pltpu.make_async_copy(bufs[b],
                                     out_hbm.at[pl.ds(base + off, CHUNK)],
                                     osems[b])

    for b in range(LAG):
        gat(b, b).start()

    @pl.loop(0, N_CHUNKS, step=NBUF)
    def _(j):
        for b in range(NBUF):
            c = j + b
            gat(c, b).wait()
            put(c, b).start()
            nb = (b + LAG) % NBUF
            cw = c - FREE

            @pl.when(cw >= 0)
            def _():
                put(cw, nb).wait()

            cg = c + LAG

            @pl.when(cg < N_CHUNKS)
            def _():
                gat(cg, nb).start()

    for k in range(N_CHUNKS - FREE, N_CHUNKS):
        put(k, k % NBUF).wait()


def kernel(input_ids, speaker_ids, char_emb, spk_table, fc_w, fc_b, ln_gamma,
           ln_beta):
    ln_table, spk_feat = _prep(char_emb, ln_gamma.reshape(1, H),
                               ln_beta.reshape(1, H), speaker_ids, spk_table,
                               fc_w, fc_b.reshape(1, H))
    flat = _sc_gather(ln_table, input_ids.reshape(TOKENS))
    return flat.reshape(B, L, H), spk_feat.reshape(B, 1, H)
